# baseline (speedup 1.0000x reference)
import functools

import jax
import jax.numpy as jnp
from jax import lax
from jax.experimental import pallas as pl
from jax.experimental.pallas import tpu as pltpu
from jax.experimental.pallas import tpu_sc as plsc

N_NODES = 10000
N_EDGES = 320000
D = 128
NUM_MOL = 512

NPAD = 10240
NBLK = NPAD // 128
NC = 2
NS = 16
NW = NC * NS
CHUNK = 128
KCH = 80
EPW = KCH * CHUNK
ROWS_PER_TILE = NPAD // NS

f32 = jnp.float32


def _prep_body(x_ref, atomp_ref, wia_ref, wua_ref, bondp_ref, wib_ref,
               binit_ref, nu_ref, m0_ref):
    xb = x_ref[0, 0, :]
    col = lax.broadcasted_iota(jnp.int32, (128, 128), 1)
    oh = (xb[:, None] == col).astype(f32)
    atoms = jnp.dot(oh, atomp_ref[...], preferred_element_type=f32)
    ni = jnp.dot(atoms, wia_ref[...], preferred_element_type=f32) + binit_ref[0:1, :]
    nu_ref[...] = jnp.dot(atoms, wua_ref[...], preferred_element_type=f32)
    bi = jnp.dot(bondp_ref[...], wib_ref[...], preferred_element_type=f32)
    for a in range(4):
        m0_ref[a] = jnp.maximum(ni + bi[a:a + 1, :], 0.0)


def _mid_body(p_ref, nu_ref, wc_ref, bondp_ref, wub_ref, bupd_ref, m_ref):
    agg = p_ref[0] + p_ref[1]
    h = nu_ref[...] + jnp.dot(agg, wc_ref[...], preferred_element_type=f32)
    bu = jnp.dot(bondp_ref[...], wub_ref[...], preferred_element_type=f32) \
        + bupd_ref[0:1, :]
    for a in range(4):
        m_ref[a] = jnp.maximum(h + bu[a:a + 1, :], 0.0)


def _final_body(p_ref, batch_ref, wro1_ref, bro1_ref, wro2_ref, bro2_ref,
                out_ref, mol_acc):
    i = pl.program_id(0)

    @pl.when(i == 0)
    def _():
        mol_acc[...] = jnp.zeros((NUM_MOL, 128), f32)

    node = p_ref[0] + p_ref[1]
    bb = batch_ref[0, 0, :]
    row = lax.broadcasted_iota(jnp.int32, (NUM_MOL, 128), 0)
    oh = (row == bb[None, :]).astype(f32)
    mol_acc[...] += jnp.dot(oh, node, preferred_element_type=f32)

    @pl.when(i == NBLK - 1)
    def _():
        h = jnp.maximum(
            jnp.dot(mol_acc[...], wro1_ref[...], preferred_element_type=f32)
            + bro1_ref[0:1, :], 0.0)
        out_ref[...] = jnp.dot(h, wro2_ref[...], preferred_element_type=f32) \
            + bro2_ref[0:1, :]


def _tc_prep(x3, atomp, wia, wua, bondp, wib, binit):
    return pl.pallas_call(
        _prep_body,
        grid=(NBLK,),
        in_specs=[
            pl.BlockSpec((1, 1, 128), lambda i: (i, 0, 0)),
            pl.BlockSpec((128, 128), lambda i: (0, 0)),
            pl.BlockSpec((128, 128), lambda i: (0, 0)),
            pl.BlockSpec((128, 128), lambda i: (0, 0)),
            pl.BlockSpec((8, 16), lambda i: (0, 0)),
            pl.BlockSpec((16, 128), lambda i: (0, 0)),
            pl.BlockSpec((8, 128), lambda i: (0, 0)),
        ],
        out_specs=[
            pl.BlockSpec((128, 128), lambda i: (i, 0)),
            pl.BlockSpec((4, 128, 128), lambda i: (0, i, 0)),
        ],
        out_shape=[
            jax.ShapeDtypeStruct((NPAD, 128), f32),
            jax.ShapeDtypeStruct((4, NPAD, 128), f32),
        ],
    )(x3, atomp, wia, wua, bondp, wib, binit)


def _tc_mid(p, nu, wc, bondp, wub, bupd):
    return pl.pallas_call(
        _mid_body,
        grid=(NBLK,),
        in_specs=[
            pl.BlockSpec((2, 128, 128), lambda i: (0, i, 0)),
            pl.BlockSpec((128, 128), lambda i: (i, 0)),
            pl.BlockSpec((128, 128), lambda i: (0, 0)),
            pl.BlockSpec((8, 16), lambda i: (0, 0)),
            pl.BlockSpec((16, 128), lambda i: (0, 0)),
            pl.BlockSpec((8, 128), lambda i: (0, 0)),
        ],
        out_specs=pl.BlockSpec((4, 128, 128), lambda i: (0, i, 0)),
        out_shape=jax.ShapeDtypeStruct((4, NPAD, 128), f32),
    )(p, nu, wc, bondp, wub, bupd)


def _tc_final(p, batch3, wro1, bro1, wro2p, bro2p):
    return pl.pallas_call(
        _final_body,
        grid=(NBLK,),
        in_specs=[
            pl.BlockSpec((2, 128, 128), lambda i: (0, i, 0)),
            pl.BlockSpec((1, 1, 128), lambda i: (i, 0, 0)),
            pl.BlockSpec((128, 256), lambda i: (0, 0)),
            pl.BlockSpec((8, 256), lambda i: (0, 0)),
            pl.BlockSpec((256, 128), lambda i: (0, 0)),
            pl.BlockSpec((8, 128), lambda i: (0, 0)),
        ],
        out_specs=pl.BlockSpec((NUM_MOL, 128), lambda i: (0, 0)),
        out_shape=jax.ShapeDtypeStruct((NUM_MOL, 128), f32),
        scratch_shapes=[pltpu.VMEM((NUM_MOL, 128), f32)],
    )(p, batch3, wro1, bro1, wro2p, bro2p)


def _sweep_body(m_hbm, g_hbm, dst_hbm, zeros_hbm, out_hbm,
                g_v, dst_v, rows_v, agg_sh, gsem):
    cid = lax.axis_index("c")
    sid = lax.axis_index("s")
    wid = sid * NC + cid

    r0 = sid * ROWS_PER_TILE
    pltpu.sync_copy(zeros_hbm.at[pl.ds(r0, ROWS_PER_TILE)],
                    agg_sh.at[pl.ds(r0, ROWS_PER_TILE)])
    plsc.subcore_barrier()

    pltpu.sync_copy(g_hbm.at[wid], g_v)
    pltpu.sync_copy(dst_hbm.at[wid], dst_v)

    def body(j, carry):
        pltpu.async_copy(m_hbm.at[g_v.at[j]], rows_v, gsem).wait()
        pltpu.sync_copy(rows_v, agg_sh.at[dst_v.at[j]], add=True)
        return carry

    lax.fori_loop(0, KCH, body, 0)

    plsc.subcore_barrier()

    pltpu.sync_copy(agg_sh.at[pl.ds(r0, ROWS_PER_TILE)],
                    out_hbm.at[pl.ds(cid * NPAD + r0, ROWS_PER_TILE)])


@functools.cache
def _sc_sweep_fn():
    return pl.kernel(
        _sweep_body,
        out_type=jax.ShapeDtypeStruct((NC * NPAD, 128), f32),
        mesh=plsc.VectorSubcoreMesh(core_axis_name="c", subcore_axis_name="s"),
        scratch_types=[
            pltpu.VMEM((KCH, CHUNK), jnp.int32),
            pltpu.VMEM((KCH, CHUNK), jnp.int32),
            pltpu.VMEM((CHUNK, 128), f32),
            pltpu.VMEM_SHARED((NPAD, 128), f32),
            pltpu.SemaphoreType.DMA,
        ],
    )


def kernel(x, edge_attr, edge_index, batch, atom_table, bond_table,
           W_init, b_init, W_upd, b_upd, W_ro1, b_ro1, W_ro2, b_ro2):
    x = x.astype(jnp.int32)
    ea = edge_attr.astype(jnp.int32)
    src = edge_index[0].astype(jnp.int32)
    dst = edge_index[1].astype(jnp.int32)
    batch = batch.astype(jnp.int32)

    x3 = jnp.pad(x, (0, NPAD - N_NODES), constant_values=127).reshape(NBLK, 1, 128)
    batch3 = jnp.pad(batch, (0, NPAD - N_NODES),
                     constant_values=NUM_MOL + 7).reshape(NBLK, 1, 128)

    atomp = jnp.zeros((128, 128), f32).at[:119].set(atom_table.astype(f32))
    bondp = jnp.zeros((8, 16), f32).at[:4].set(bond_table.astype(f32))
    wia, wib = W_init[:128], W_init[128:144]
    wua, wub, wc = W_upd[:128], W_upd[128:144], W_upd[144:272]
    binit = jnp.broadcast_to(b_init[None, :], (8, 128))
    bupd = jnp.broadcast_to(b_upd[None, :], (8, 128))
    bro1 = jnp.broadcast_to(b_ro1[None, :], (8, 256))
    wro2p = jnp.pad(W_ro2, ((0, 0), (0, 127)))
    bro2p = jnp.broadcast_to(
        jnp.pad(b_ro2[None, :], ((0, 0), (0, 127))), (8, 128))

    g = ea * NPAD + src
    g3 = jnp.pad(g, (0, NW * EPW - N_EDGES)).reshape(NW, KCH, CHUNK)
    trash = N_NODES + jnp.arange(NW * EPW - N_EDGES, dtype=jnp.int32) \
        % (NPAD - N_NODES)
    dst3 = jnp.concatenate([dst, trash]).reshape(NW, KCH, CHUNK)
    zeros_rows = jnp.zeros((NPAD, 128), f32)

    nu, m = _tc_prep(x3, atomp, wia, wua, bondp, wib, binit)

    for _ in range(3):
        p = _sc_sweep_fn()(m.reshape(4 * NPAD, 128), g3, dst3, zeros_rows)
        m = _tc_mid(p.reshape(NC, NPAD, 128), nu, wc, bondp, wub, bupd)

    p = _sc_sweep_fn()(m.reshape(4 * NPAD, 128), g3, dst3, zeros_rows)
    out_mat = _tc_final(p.reshape(NC, NPAD, 128), batch3,
                        W_ro1, bro1, wro2p, bro2p)
    return out_mat[:, 0]


# speedup vs baseline: 1.4996x; 1.4996x over previous
import functools

import jax
import jax.numpy as jnp
from jax import lax
from jax.experimental import pallas as pl
from jax.experimental.pallas import tpu as pltpu
from jax.experimental.pallas import tpu_sc as plsc

N_NODES = 10000
N_EDGES = 320000
D = 128
NUM_MOL = 512

NPAD = 10240
NBLK = NPAD // 128
NC = 2
NS = 16
NW = NC * NS
CHUNK = 128
KCH = 79
EPW = KCH * CHUNK
ROWS_PER_TILE = NPAD // NS

f32 = jnp.float32


def _prep_body(x_ref, atomp_ref, wia_ref, wua_ref, bondp_ref, wib_ref,
               binit_ref, nu_ref, m0_ref):
    xb = x_ref[0, 0, :]
    col = lax.broadcasted_iota(jnp.int32, (128, 128), 1)
    oh = (xb[:, None] == col).astype(f32)
    atoms = jnp.dot(oh, atomp_ref[...], preferred_element_type=f32)
    ni = jnp.dot(atoms, wia_ref[...], preferred_element_type=f32) + binit_ref[0:1, :]
    nu_ref[...] = jnp.dot(atoms, wua_ref[...], preferred_element_type=f32)
    bi = jnp.dot(bondp_ref[...], wib_ref[...], preferred_element_type=f32)
    for a in range(4):
        m0_ref[a] = jnp.maximum(ni + bi[a:a + 1, :], 0.0)


def _mid_body(p_ref, nu_ref, wc_ref, bondp_ref, wub_ref, bupd_ref, m_ref):
    agg = p_ref[0] + p_ref[1]
    h = nu_ref[...] + jnp.dot(agg, wc_ref[...], preferred_element_type=f32)
    bu = jnp.dot(bondp_ref[...], wub_ref[...], preferred_element_type=f32) \
        + bupd_ref[0:1, :]
    for a in range(4):
        m_ref[a] = jnp.maximum(h + bu[a:a + 1, :], 0.0)


def _final_body(p_ref, batch_ref, wro1_ref, bro1_ref, wro2_ref, bro2_ref,
                out_ref, mol_acc):
    i = pl.program_id(0)

    @pl.when(i == 0)
    def _():
        mol_acc[...] = jnp.zeros((NUM_MOL, 128), f32)

    node = p_ref[0] + p_ref[1]
    bb = batch_ref[0, 0, :]
    row = lax.broadcasted_iota(jnp.int32, (NUM_MOL, 128), 0)
    oh = (row == bb[None, :]).astype(f32)
    mol_acc[...] += jnp.dot(oh, node, preferred_element_type=f32)

    @pl.when(i == NBLK - 1)
    def _():
        h = jnp.maximum(
            jnp.dot(mol_acc[...], wro1_ref[...], preferred_element_type=f32)
            + bro1_ref[0:1, :], 0.0)
        out_ref[...] = jnp.dot(h, wro2_ref[...], preferred_element_type=f32) \
            + bro2_ref[0:1, :]


def _tc_prep(x3, atomp, wia, wua, bondp, wib, binit):
    return pl.pallas_call(
        _prep_body,
        grid=(NBLK,),
        in_specs=[
            pl.BlockSpec((1, 1, 128), lambda i: (i, 0, 0)),
            pl.BlockSpec((128, 128), lambda i: (0, 0)),
            pl.BlockSpec((128, 128), lambda i: (0, 0)),
            pl.BlockSpec((128, 128), lambda i: (0, 0)),
            pl.BlockSpec((8, 16), lambda i: (0, 0)),
            pl.BlockSpec((16, 128), lambda i: (0, 0)),
            pl.BlockSpec((8, 128), lambda i: (0, 0)),
        ],
        out_specs=[
            pl.BlockSpec((128, 128), lambda i: (i, 0)),
            pl.BlockSpec((4, 128, 128), lambda i: (0, i, 0)),
        ],
        out_shape=[
            jax.ShapeDtypeStruct((NPAD, 128), f32),
            jax.ShapeDtypeStruct((4, NPAD, 128), f32),
        ],
    )(x3, atomp, wia, wua, bondp, wib, binit)


def _tc_mid(p, nu, wc, bondp, wub, bupd):
    return pl.pallas_call(
        _mid_body,
        grid=(NBLK,),
        in_specs=[
            pl.BlockSpec((2, 128, 128), lambda i: (0, i, 0)),
            pl.BlockSpec((128, 128), lambda i: (i, 0)),
            pl.BlockSpec((128, 128), lambda i: (0, 0)),
            pl.BlockSpec((8, 16), lambda i: (0, 0)),
            pl.BlockSpec((16, 128), lambda i: (0, 0)),
            pl.BlockSpec((8, 128), lambda i: (0, 0)),
        ],
        out_specs=pl.BlockSpec((4, 128, 128), lambda i: (0, i, 0)),
        out_shape=jax.ShapeDtypeStruct((4, NPAD, 128), f32),
    )(p, nu, wc, bondp, wub, bupd)


def _tc_final(p, batch3, wro1, bro1, wro2p, bro2p):
    return pl.pallas_call(
        _final_body,
        grid=(NBLK,),
        in_specs=[
            pl.BlockSpec((2, 128, 128), lambda i: (0, i, 0)),
            pl.BlockSpec((1, 1, 128), lambda i: (i, 0, 0)),
            pl.BlockSpec((128, 256), lambda i: (0, 0)),
            pl.BlockSpec((8, 256), lambda i: (0, 0)),
            pl.BlockSpec((256, 128), lambda i: (0, 0)),
            pl.BlockSpec((8, 128), lambda i: (0, 0)),
        ],
        out_specs=pl.BlockSpec((NUM_MOL, 128), lambda i: (0, 0)),
        out_shape=jax.ShapeDtypeStruct((NUM_MOL, 128), f32),
        scratch_shapes=[pltpu.VMEM((NUM_MOL, 128), f32)],
    )(p, batch3, wro1, bro1, wro2p, bro2p)


def _sweep_body(m_hbm, g_hbm, dst_hbm, zeros_hbm, out_hbm,
                g_v, dst_v, rows_v, agg_sh, gsem):
    cid = lax.axis_index("c")
    sid = lax.axis_index("s")
    wid = sid * NC + cid

    r0 = sid * ROWS_PER_TILE
    pltpu.sync_copy(zeros_hbm.at[pl.ds(r0, ROWS_PER_TILE)],
                    agg_sh.at[pl.ds(r0, ROWS_PER_TILE)])
    plsc.subcore_barrier()

    pltpu.sync_copy(g_hbm.at[wid], g_v)
    pltpu.sync_copy(dst_hbm.at[wid], dst_v)

    def body(j, carry):
        pltpu.async_copy(m_hbm.at[g_v.at[j]], rows_v, gsem).wait()
        pltpu.sync_copy(rows_v, agg_sh.at[dst_v.at[j]], add=True)
        return carry

    lax.fori_loop(0, KCH, body, 0)

    plsc.subcore_barrier()

    pltpu.sync_copy(agg_sh.at[pl.ds(r0, ROWS_PER_TILE)],
                    out_hbm.at[pl.ds(cid * NPAD + r0, ROWS_PER_TILE)])


@functools.cache
def _sc_sweep_fn():
    return pl.kernel(
        _sweep_body,
        out_type=jax.ShapeDtypeStruct((NC * NPAD, 128), f32),
        mesh=plsc.VectorSubcoreMesh(core_axis_name="c", subcore_axis_name="s"),
        scratch_types=[
            pltpu.VMEM((KCH, CHUNK), jnp.int32),
            pltpu.VMEM((KCH, CHUNK), jnp.int32),
            pltpu.VMEM((CHUNK, 128), f32),
            pltpu.VMEM_SHARED((NPAD, 128), f32),
            pltpu.SemaphoreType.DMA,
        ],
    )


def kernel(x, edge_attr, edge_index, batch, atom_table, bond_table,
           W_init, b_init, W_upd, b_upd, W_ro1, b_ro1, W_ro2, b_ro2):
    x = x.astype(jnp.int32)
    ea = edge_attr.astype(jnp.int32)
    src = edge_index[0].astype(jnp.int32)
    dst = edge_index[1].astype(jnp.int32)
    batch = batch.astype(jnp.int32)

    x3 = jnp.pad(x, (0, NPAD - N_NODES), constant_values=127).reshape(NBLK, 1, 128)
    batch3 = jnp.pad(batch, (0, NPAD - N_NODES),
                     constant_values=NUM_MOL + 7).reshape(NBLK, 1, 128)

    atomp = jnp.zeros((128, 128), f32).at[:119].set(atom_table.astype(f32))
    bondp = jnp.zeros((8, 16), f32).at[:4].set(bond_table.astype(f32))
    wia, wib = W_init[:128], W_init[128:144]
    wua, wub, wc = W_upd[:128], W_upd[128:144], W_upd[144:272]
    binit = jnp.broadcast_to(b_init[None, :], (8, 128))
    bupd = jnp.broadcast_to(b_upd[None, :], (8, 128))
    bro1 = jnp.broadcast_to(b_ro1[None, :], (8, 256))
    wro2p = jnp.pad(W_ro2, ((0, 0), (0, 127)))
    bro2p = jnp.broadcast_to(
        jnp.pad(b_ro2[None, :], ((0, 0), (0, 127))), (8, 128))

    g = ea * NPAD + src
    g3 = jnp.pad(g, (0, NW * EPW - N_EDGES)).reshape(NW, KCH, CHUNK)
    trash = N_NODES + jnp.arange(NW * EPW - N_EDGES, dtype=jnp.int32) \
        % (NPAD - N_NODES)
    dst3 = jnp.concatenate([dst, trash]).reshape(NW, KCH, CHUNK)
    zeros_rows = jnp.zeros((NPAD, 128), f32)

    nu, m = _tc_prep(x3, atomp, wia, wua, bondp, wib, binit)

    for _ in range(3):
        p = _sc_sweep_fn()(m.reshape(4 * NPAD, 128), g3, dst3, zeros_rows)
        m = _tc_mid(p.reshape(NC, NPAD, 128), nu, wc, bondp, wub, bupd)

    p = _sc_sweep_fn()(m.reshape(4 * NPAD, 128), g3, dst3, zeros_rows)
    out_mat = _tc_final(p.reshape(NC, NPAD, 128), batch3,
                        W_ro1, bro1, wro2p, bro2p)
    return out_mat[:, 0]


# speedup vs baseline: 1.7878x; 1.1922x over previous
import functools

import jax
import jax.numpy as jnp
from jax import lax
from jax.experimental import pallas as pl
from jax.experimental.pallas import tpu as pltpu
from jax.experimental.pallas import tpu_sc as plsc

N_NODES = 10000
N_EDGES = 320000
D = 128
NUM_MOL = 512

NPAD = 10240
NBLK = NPAD // 128
NC = 2
NS = 16
NW = NC * NS
CHUNK = 128
KCH = 79
NBUF = 2
EPW = KCH * CHUNK
ROWS_PER_TILE = NPAD // NS

f32 = jnp.float32
bf16 = jnp.bfloat16


def _prep_body(x_ref, atomp_ref, wia_ref, wua_ref, bondp_ref, wib_ref,
               binit_ref, nu_ref, m0_ref):
    xb = x_ref[0, 0, :]
    col = lax.broadcasted_iota(jnp.int32, (128, 128), 1)
    oh = (xb[:, None] == col).astype(f32)
    atoms = jnp.dot(oh, atomp_ref[...], preferred_element_type=f32)
    ni = jnp.dot(atoms, wia_ref[...], preferred_element_type=f32) + binit_ref[0:1, :]
    nu_ref[...] = jnp.dot(atoms, wua_ref[...], preferred_element_type=f32)
    bi = jnp.dot(bondp_ref[...], wib_ref[...], preferred_element_type=f32)
    for a in range(4):
        m0_ref[a] = jnp.maximum(ni + bi[a:a + 1, :], 0.0)


def _mid_body(p_ref, nu_ref, wc_ref, bondp_ref, wub_ref, bupd_ref, m_ref):
    agg = p_ref[0].astype(f32) + p_ref[1].astype(f32)
    h = nu_ref[...] + jnp.dot(agg, wc_ref[...], preferred_element_type=f32)
    bu = jnp.dot(bondp_ref[...], wub_ref[...], preferred_element_type=f32) \
        + bupd_ref[0:1, :]
    for a in range(4):
        m_ref[a] = jnp.maximum(h + bu[a:a + 1, :], 0.0)


def _final_body(p_ref, batch_ref, wro1_ref, bro1_ref, wro2_ref, bro2_ref,
                out_ref, mol_acc):
    i = pl.program_id(0)

    @pl.when(i == 0)
    def _():
        mol_acc[...] = jnp.zeros((NUM_MOL, 128), f32)

    node = p_ref[0].astype(f32) + p_ref[1].astype(f32)
    bb = batch_ref[0, 0, :]
    row = lax.broadcasted_iota(jnp.int32, (NUM_MOL, 128), 0)
    oh = (row == bb[None, :]).astype(f32)
    mol_acc[...] += jnp.dot(oh, node, preferred_element_type=f32)

    @pl.when(i == NBLK - 1)
    def _():
        h = jnp.maximum(
            jnp.dot(mol_acc[...], wro1_ref[...], preferred_element_type=f32)
            + bro1_ref[0:1, :], 0.0)
        out_ref[...] = jnp.dot(h, wro2_ref[...], preferred_element_type=f32) \
            + bro2_ref[0:1, :]


def _tc_prep(x3, atomp, wia, wua, bondp, wib, binit):
    return pl.pallas_call(
        _prep_body,
        grid=(NBLK,),
        in_specs=[
            pl.BlockSpec((1, 1, 128), lambda i: (i, 0, 0)),
            pl.BlockSpec((128, 128), lambda i: (0, 0)),
            pl.BlockSpec((128, 128), lambda i: (0, 0)),
            pl.BlockSpec((128, 128), lambda i: (0, 0)),
            pl.BlockSpec((8, 16), lambda i: (0, 0)),
            pl.BlockSpec((16, 128), lambda i: (0, 0)),
            pl.BlockSpec((8, 128), lambda i: (0, 0)),
        ],
        out_specs=[
            pl.BlockSpec((128, 128), lambda i: (i, 0)),
            pl.BlockSpec((4, 128, 128), lambda i: (0, i, 0)),
        ],
        out_shape=[
            jax.ShapeDtypeStruct((NPAD, 128), f32),
            jax.ShapeDtypeStruct((4, NPAD, 128), f32),
        ],
    )(x3, atomp, wia, wua, bondp, wib, binit)


def _tc_mid(p, nu, wc, bondp, wub, bupd):
    return pl.pallas_call(
        _mid_body,
        grid=(NBLK,),
        in_specs=[
            pl.BlockSpec((2, 128, 128), lambda i: (0, i, 0)),
            pl.BlockSpec((128, 128), lambda i: (i, 0)),
            pl.BlockSpec((128, 128), lambda i: (0, 0)),
            pl.BlockSpec((8, 16), lambda i: (0, 0)),
            pl.BlockSpec((16, 128), lambda i: (0, 0)),
            pl.BlockSpec((8, 128), lambda i: (0, 0)),
        ],
        out_specs=pl.BlockSpec((4, 128, 128), lambda i: (0, i, 0)),
        out_shape=jax.ShapeDtypeStruct((4, NPAD, 128), f32),
    )(p, nu, wc, bondp, wub, bupd)


def _tc_final(p, batch3, wro1, bro1, wro2p, bro2p):
    return pl.pallas_call(
        _final_body,
        grid=(NBLK,),
        in_specs=[
            pl.BlockSpec((2, 128, 128), lambda i: (0, i, 0)),
            pl.BlockSpec((1, 1, 128), lambda i: (i, 0, 0)),
            pl.BlockSpec((128, 256), lambda i: (0, 0)),
            pl.BlockSpec((8, 256), lambda i: (0, 0)),
            pl.BlockSpec((256, 128), lambda i: (0, 0)),
            pl.BlockSpec((8, 128), lambda i: (0, 0)),
        ],
        out_specs=pl.BlockSpec((NUM_MOL, 128), lambda i: (0, 0)),
        out_shape=jax.ShapeDtypeStruct((NUM_MOL, 128), f32),
        scratch_shapes=[pltpu.VMEM((NUM_MOL, 128), f32)],
    )(p, batch3, wro1, bro1, wro2p, bro2p)


def _sweep_body(m_hbm, gd_hbm, zeros_hbm, out_hbm,
                gd_v, gidx, didx, rows_v, agg_sh, gsems, ssems):
    cid = lax.axis_index("c")
    sid = lax.axis_index("s")
    wid = sid * NC + cid

    r0 = sid * ROWS_PER_TILE
    pltpu.sync_copy(zeros_hbm.at[pl.ds(r0, ROWS_PER_TILE)],
                    agg_sh.at[pl.ds(r0, ROWS_PER_TILE)])
    plsc.subcore_barrier()

    pltpu.sync_copy(gd_hbm.at[wid], gd_v)

    def unpack(j, b):
        for k in range(CHUNK // 16):
            w = gd_v[j, pl.ds(k * 16, 16)]
            gidx[b, pl.ds(k * 16, 16)] = lax.shift_right_logical(w, 16)
            didx[b, pl.ds(k * 16, 16)] = w & 0xFFFF

    for b in range(NBUF):
        unpack(b, b)
        pltpu.async_copy(m_hbm.at[gidx.at[b]], rows_v.at[b], gsems.at[b])

    def stage(j, b, prefetch):
        pltpu.make_async_copy(m_hbm.at[gidx.at[b]], rows_v.at[b],
                              gsems.at[b]).wait()
        pltpu.async_copy(rows_v.at[b], agg_sh.at[didx.at[b]],
                         ssems.at[b], add=True).wait()
        if prefetch:
            unpack(j + NBUF, b)
            pltpu.async_copy(m_hbm.at[gidx.at[b]], rows_v.at[b],
                             gsems.at[b])

    def outer(i, carry):
        for b in range(NBUF):
            stage(i * NBUF + b, b, True)
        return carry

    lax.fori_loop(0, (KCH - 3) // NBUF, outer, 0)
    stage(KCH - 3, (KCH - 3) % NBUF, True)
    stage(KCH - 2, (KCH - 2) % NBUF, False)
    stage(KCH - 1, (KCH - 1) % NBUF, False)

    plsc.subcore_barrier()

    pltpu.sync_copy(agg_sh.at[pl.ds(r0, ROWS_PER_TILE)],
                    out_hbm.at[pl.ds(cid * NPAD + r0, ROWS_PER_TILE)])


@functools.cache
def _sc_sweep_fn():
    return pl.kernel(
        _sweep_body,
        out_type=jax.ShapeDtypeStruct((NC * NPAD, 128), f32),
        mesh=plsc.VectorSubcoreMesh(core_axis_name="c", subcore_axis_name="s"),
        scratch_types=[
            pltpu.VMEM((KCH, CHUNK), jnp.int32),
            pltpu.VMEM((NBUF, CHUNK), jnp.int32),
            pltpu.VMEM((NBUF, CHUNK), jnp.int32),
            pltpu.VMEM((NBUF, CHUNK, 128), f32),
            pltpu.VMEM_SHARED((NPAD, 128), f32),
            pltpu.SemaphoreType.DMA((NBUF,)),
            pltpu.SemaphoreType.DMA((NBUF,)),
        ],
    )


def kernel(x, edge_attr, edge_index, batch, atom_table, bond_table,
           W_init, b_init, W_upd, b_upd, W_ro1, b_ro1, W_ro2, b_ro2):
    x = x.astype(jnp.int32)
    ea = edge_attr.astype(jnp.int32)
    src = edge_index[0].astype(jnp.int32)
    dst = edge_index[1].astype(jnp.int32)
    batch = batch.astype(jnp.int32)

    x3 = jnp.pad(x, (0, NPAD - N_NODES), constant_values=127).reshape(NBLK, 1, 128)
    batch3 = jnp.pad(batch, (0, NPAD - N_NODES),
                     constant_values=NUM_MOL + 7).reshape(NBLK, 1, 128)

    atomp = jnp.zeros((128, 128), f32).at[:119].set(atom_table.astype(f32))
    bondp = jnp.zeros((8, 16), f32).at[:4].set(bond_table.astype(f32))
    wia, wib = W_init[:128], W_init[128:144]
    wua, wub, wc = W_upd[:128], W_upd[128:144], W_upd[144:272]
    binit = jnp.broadcast_to(b_init[None, :], (8, 128))
    bupd = jnp.broadcast_to(b_upd[None, :], (8, 128))
    bro1 = jnp.broadcast_to(b_ro1[None, :], (8, 256))
    wro2p = jnp.pad(W_ro2, ((0, 0), (0, 127)))
    bro2p = jnp.broadcast_to(
        jnp.pad(b_ro2[None, :], ((0, 0), (0, 127))), (8, 128))

    g = ea * NPAD + src
    trash = N_NODES + jnp.arange(NW * EPW - N_EDGES, dtype=jnp.int32) \
        % (NPAD - N_NODES)
    gfull = jnp.pad(g, (0, NW * EPW - N_EDGES))
    dfull = jnp.concatenate([dst, trash])
    gd3 = ((gfull << 16) | dfull).reshape(NW, KCH, CHUNK)
    zeros_rows = jnp.zeros((NPAD, 128), f32)

    nu, m = _tc_prep(x3, atomp, wia, wua, bondp, wib, binit)

    for _ in range(3):
        p = _sc_sweep_fn()(m.reshape(4 * NPAD, 128), gd3, zeros_rows)
        m = _tc_mid(p.reshape(NC, NPAD, 128), nu, wc, bondp, wub, bupd)

    p = _sc_sweep_fn()(m.reshape(4 * NPAD, 128), gd3, zeros_rows)
    out_mat = _tc_final(p.reshape(NC, NPAD, 128), batch3,
                        W_ro1, bro1, wro2p, bro2p)
    return out_mat[:, 0]
